# baseline (speedup 1.0000x reference)
import functools

import jax
import jax.numpy as jnp
from jax import lax
from jax.experimental import pallas as pl
from jax.experimental.pallas import tpu as pltpu
from jax.experimental.pallas import tpu_sc as plsc

K = 8192
D = 32
NW = 32


def _dist_kernel(z_ref, cb_ref, d_ref, idx_ref, minv_ref, mini_ref, *, T, KB):
    k = pl.program_id(1)
    z = z_ref[0]
    cbm2 = cb_ref[...] * -2.0
    zsq = jnp.sum(z * z, axis=0, keepdims=True)
    esq = jnp.sum(cbm2 * cbm2, axis=1, keepdims=True) * 0.25
    mm2 = jnp.dot(cbm2, z, preferred_element_type=jnp.float32)
    d = (zsq + esq) + mm2
    d_ref[0] = d

    bmin = jnp.min(d, axis=0, keepdims=True)
    rows = jax.lax.broadcasted_iota(jnp.int32, (KB, T), 0)
    loc = jnp.min(jnp.where(d == bmin, rows, K), axis=0, keepdims=True)
    cand = loc + k * KB

    @pl.when(k == 0)
    def _():
        minv_ref[...] = bmin
        mini_ref[...] = cand

    @pl.when(k > 0)
    def _():
        better = bmin < minv_ref[...]
        minv_ref[...] = jnp.where(better, bmin, minv_ref[...])
        mini_ref[...] = jnp.where(better, cand, mini_ref[...])

    @pl.when(k == pl.num_programs(1) - 1)
    def _():
        idx_ref[0] = mini_ref[...]


def _distance_pass(z_bct, cb):
    B, _, T = z_bct.shape
    KB = min(K, (4 * 1024 * 1024) // T)
    NK = K // KB
    return pl.pallas_call(
        functools.partial(_dist_kernel, T=T, KB=KB),
        grid=(B, NK),
        in_specs=[
            pl.BlockSpec((1, D, T), lambda b, k: (b, 0, 0)),
            pl.BlockSpec((KB, D), lambda b, k: (k, 0)),
        ],
        out_specs=[
            pl.BlockSpec((1, KB, T), lambda b, k: (b, k, 0)),
            pl.BlockSpec((1, 1, T), lambda b, k: (b, 0, 0)),
        ],
        out_shape=[
            jax.ShapeDtypeStruct((B, K, T), jnp.float32),
            jax.ShapeDtypeStruct((B, 1, T), jnp.int32),
        ],
        scratch_shapes=[
            pltpu.VMEM((1, T), jnp.float32),
            pltpu.VMEM((1, T), jnp.int32),
        ],
    )(z_bct, cb)


_N_CHUNK = 128
_NJ = _N_CHUNK // 16
_W_OFF = (0, 16, 8)


def _make_sc_gather(B, T):
    n = _N_CHUNK
    nc = (B * T) // n
    wpb = T // n
    mesh = plsc.VectorSubcoreMesh(core_axis_name="c", subcore_axis_name="s")

    @functools.partial(
        pl.kernel, mesh=mesh,
        compiler_params=pltpu.CompilerParams(needs_layout_passes=False,
                                             use_tc_tiling_on_sc=False),
        out_type=[
            jax.ShapeDtypeStruct((B, D, T), jnp.float32),
            jax.ShapeDtypeStruct((B, D, T), jnp.float32),
            jax.ShapeDtypeStruct((nc, 1, K), jnp.float32),
        ],
        scratch_types=[
            pltpu.VMEM((n,), jnp.int32),
            pltpu.VMEM((n, D), jnp.float32),
            pltpu.VMEM((D, n), jnp.float32),
            pltpu.VMEM((D, n), jnp.float32),
            pltpu.VMEM((D, n), jnp.float32),
            pltpu.VMEM((K,), jnp.float32),
            pltpu.SemaphoreType.DMA,
        ],
    )
    def sc_kernel(cb_hbm, idx_hbm, z_hbm, q_hbm, zq_hbm, cnt_hbm,
                  idx_v, rows_v, q_v, zq_v, z_v, cnt_v, sem):
        wid = lax.axis_index("s") * 2 + lax.axis_index("c")

        @pl.when(wid < nc)
        def _():
            b = wid // wpb
            t0 = (wid % wpb) * n
            base = wid * n

            pltpu.sync_copy(idx_hbm.at[pl.ds(base, n)], idx_v)
            gat = pltpu.async_copy(cb_hbm.at[idx_v], rows_v, sem)
            pltpu.sync_copy(z_hbm.at[b, :, pl.ds(t0, n)], z_v)
            gat.wait()

            lane = lax.iota(jnp.int32, 16)

            def jb(j, _):
                rowsel = lane + j * 16
                for c in range(D):
                    v = plsc.load_gather(rows_v,
                                         [rowsel, jnp.full((16,), c, jnp.int32)])
                    zv = z_v[c, pl.ds(j * 16, 16)]
                    q_v[c, pl.ds(j * 16, 16)] = v
                    zq_v[c, pl.ds(j * 16, 16)] = zv + (v - zv)
                return 0

            lax.fori_loop(0, _NJ, jb, 0)

            def zbody(i, _):
                cnt_v[pl.ds(i * 16, 16)] = jnp.zeros((16,), jnp.float32)
                return 0

            lax.fori_loop(0, K // 16, zbody, 0)

            ones = jnp.full((16,), 1.0, jnp.float32)

            def hbody(j, _):
                iv = idx_v[pl.ds(j * 16, 16)]
                plsc.addupdate_scatter(cnt_v, [iv], ones)
                return 0

            lax.fori_loop(0, _NJ, hbody, 0)

            pltpu.sync_copy(q_v, q_hbm.at[b, :, pl.ds(t0, n)])
            pltpu.sync_copy(zq_v, zq_hbm.at[b, :, pl.ds(t0, n)])
            pltpu.sync_copy(cnt_v, cnt_hbm.at[wid, 0])

    return sc_kernel


def _perp_kernel(c0_ref, c1_ref, c2_ref, p0_ref, p1_ref, p2_ref, *, ns):
    for cr, pr, nn in ((c0_ref, p0_ref, ns[0]), (c1_ref, p1_ref, ns[1]),
                       (c2_ref, p2_ref, ns[2])):
        c = jnp.sum(cr[:, 0, :], axis=0, keepdims=True)
        p = c * (1.0 / nn)
        ent = jnp.sum(p * jnp.log(p + 1e-10))
        pr[...] = jnp.full((1, 1), 1.0, jnp.float32) * jnp.exp(-ent)


def kernel(z0_bct, z1_bct, z2_bct, cb0, cb1, cb2):
    levels = ((z0_bct, cb0), (z1_bct, cb1), (z2_bct, cb2))
    zqs, qs, ds, cnts, ns = [], [], [], [], []
    for z_bct, cb in levels:
        B, _, T = z_bct.shape
        d, idx = _distance_pass(z_bct, cb)
        q, zq, cnt = _make_sc_gather(B, T)(cb, idx.reshape(B * T), z_bct)
        zqs.append(zq)
        qs.append(q)
        ds.append(d)
        cnts.append(cnt)
        ns.append(B * T)

    perps = pl.pallas_call(
        functools.partial(_perp_kernel, ns=tuple(ns)),
        grid=(1,),
        in_specs=[pl.BlockSpec(c.shape, lambda i: (0, 0, 0)) for c in cnts],
        out_specs=[pl.BlockSpec((1, 1), lambda i: (0, 0))] * 3,
        out_shape=[jax.ShapeDtypeStruct((1, 1), jnp.float32)] * 3,
    )(*cnts)

    return (*zqs, *qs, *ds, *(p.reshape(()) for p in perps))


# speedup vs baseline: 1.0210x; 1.0210x over previous
import functools

import jax
import jax.numpy as jnp
from jax import lax
from jax.experimental import pallas as pl
from jax.experimental.pallas import tpu as pltpu
from jax.experimental.pallas import tpu_sc as plsc

K = 8192
D = 32
NW = 32


def _dist_kernel(z_ref, cb_ref, d_ref, idx_ref, minv_ref, mini_ref, *, T, KB):
    k = pl.program_id(1)
    z = z_ref[0]
    cbm2 = cb_ref[...] * -2.0
    zsq = jnp.sum(z * z, axis=0, keepdims=True)
    esq = jnp.sum(cbm2 * cbm2, axis=1, keepdims=True) * 0.25
    mm2 = jnp.dot(cbm2, z, preferred_element_type=jnp.float32)
    d = (zsq + esq) + mm2
    d_ref[0] = d

    bmin = jnp.min(d, axis=0, keepdims=True)
    rows = jax.lax.broadcasted_iota(jnp.int32, (KB, T), 0)
    loc = jnp.min(jnp.where(d == bmin, rows, K), axis=0, keepdims=True)
    cand = loc + k * KB

    @pl.when(k == 0)
    def _():
        minv_ref[...] = bmin
        mini_ref[...] = cand

    @pl.when(k > 0)
    def _():
        better = bmin < minv_ref[...]
        minv_ref[...] = jnp.where(better, bmin, minv_ref[...])
        mini_ref[...] = jnp.where(better, cand, mini_ref[...])

    @pl.when(k == pl.num_programs(1) - 1)
    def _():
        idx_ref[0] = mini_ref[...]


def _distance_pass(z_bct, cb):
    B, _, T = z_bct.shape
    KB = min(K, (4 * 1024 * 1024) // T)
    NK = K // KB
    return pl.pallas_call(
        functools.partial(_dist_kernel, T=T, KB=KB),
        grid=(B, NK),
        in_specs=[
            pl.BlockSpec((1, D, T), lambda b, k: (b, 0, 0)),
            pl.BlockSpec((KB, D), lambda b, k: (k, 0)),
        ],
        out_specs=[
            pl.BlockSpec((1, KB, T), lambda b, k: (b, k, 0)),
            pl.BlockSpec((1, 1, T), lambda b, k: (b, 0, 0)),
        ],
        out_shape=[
            jax.ShapeDtypeStruct((B, K, T), jnp.float32),
            jax.ShapeDtypeStruct((B, 1, T), jnp.int32),
        ],
        scratch_shapes=[
            pltpu.VMEM((1, T), jnp.float32),
            pltpu.VMEM((1, T), jnp.int32),
        ],
    )(z_bct, cb)


_N_CHUNK = 128
_NJ = _N_CHUNK // 16
_W_OFF = (0, 16, 8)


def _make_sc_gather(B, T):
    n = _N_CHUNK
    nc = (B * T) // n
    wpb = T // n
    mesh = plsc.VectorSubcoreMesh(core_axis_name="c", subcore_axis_name="s")

    @functools.partial(
        pl.kernel, mesh=mesh,
        compiler_params=pltpu.CompilerParams(needs_layout_passes=False),
        out_type=[
            jax.ShapeDtypeStruct((B, D, T), jnp.float32),
            jax.ShapeDtypeStruct((B, D, T), jnp.float32),
            jax.ShapeDtypeStruct((nc, 1, K), jnp.float32),
        ],
        scratch_types=[
            pltpu.VMEM((n,), jnp.int32),
            pltpu.VMEM((n, 128), jnp.float32),
            pltpu.VMEM((D, n), jnp.float32),
            pltpu.VMEM((D, n), jnp.float32),
            pltpu.VMEM((D, n), jnp.float32),
            pltpu.VMEM((K,), jnp.float32),
            pltpu.SemaphoreType.DMA,
        ],
    )
    def sc_kernel(cb_hbm, idx_hbm, z_hbm, q_hbm, zq_hbm, cnt_hbm,
                  idx_v, rows_v, q_v, zq_v, z_v, cnt_v, sem):
        wid = lax.axis_index("s") * 2 + lax.axis_index("c")

        @pl.when(wid < nc)
        def _():
            b = wid // wpb
            t0 = (wid % wpb) * n
            base = wid * n

            pltpu.sync_copy(idx_hbm.at[pl.ds(base, n)], idx_v)
            gat = pltpu.async_copy(cb_hbm.at[idx_v], rows_v, sem)
            pltpu.sync_copy(z_hbm.at[b, :, pl.ds(t0, n)], z_v)
            gat.wait()

            lane = lax.iota(jnp.int32, 16)

            def jb(j, _):
                rowsel = lane + j * 16
                for c in range(D):
                    v = plsc.load_gather(rows_v,
                                         [rowsel, jnp.full((16,), c, jnp.int32)])
                    zv = z_v[c, pl.ds(j * 16, 16)]
                    q_v[c, pl.ds(j * 16, 16)] = v
                    zq_v[c, pl.ds(j * 16, 16)] = zv + (v - zv)
                return 0

            lax.fori_loop(0, _NJ, jb, 0)

            def zbody(i, _):
                cnt_v[pl.ds(i * 16, 16)] = jnp.zeros((16,), jnp.float32)
                return 0

            lax.fori_loop(0, K // 16, zbody, 0)

            ones = jnp.full((16,), 1.0, jnp.float32)

            def hbody(j, _):
                iv = idx_v[pl.ds(j * 16, 16)]
                plsc.addupdate_scatter(cnt_v, [iv], ones)
                return 0

            lax.fori_loop(0, _NJ, hbody, 0)

            pltpu.sync_copy(q_v, q_hbm.at[b, :, pl.ds(t0, n)])
            pltpu.sync_copy(zq_v, zq_hbm.at[b, :, pl.ds(t0, n)])
            pltpu.sync_copy(cnt_v, cnt_hbm.at[wid, 0])

    return sc_kernel


def _perp_kernel(c0_ref, c1_ref, c2_ref, p0_ref, p1_ref, p2_ref, *, ns):
    for cr, pr, nn in ((c0_ref, p0_ref, ns[0]), (c1_ref, p1_ref, ns[1]),
                       (c2_ref, p2_ref, ns[2])):
        c = jnp.sum(cr[:, 0, :], axis=0, keepdims=True)
        p = c * (1.0 / nn)
        ent = jnp.sum(p * jnp.log(p + 1e-10))
        pr[...] = jnp.full((1, 1), 1.0, jnp.float32) * jnp.exp(-ent)


def kernel(z0_bct, z1_bct, z2_bct, cb0, cb1, cb2):
    levels = ((z0_bct, cb0), (z1_bct, cb1), (z2_bct, cb2))
    zqs, qs, ds, cnts, ns = [], [], [], [], []
    for z_bct, cb in levels:
        B, _, T = z_bct.shape
        d, idx = _distance_pass(z_bct, cb)
        cb_pad = jnp.pad(cb, ((0, 0), (0, 128 - D)))
        q, zq, cnt = _make_sc_gather(B, T)(cb_pad, idx.reshape(B * T), z_bct)
        zqs.append(zq)
        qs.append(q)
        ds.append(d)
        cnts.append(cnt)
        ns.append(B * T)

    perps = pl.pallas_call(
        functools.partial(_perp_kernel, ns=tuple(ns)),
        grid=(1,),
        in_specs=[pl.BlockSpec(c.shape, lambda i: (0, 0, 0)) for c in cnts],
        out_specs=[pl.BlockSpec((1, 1), lambda i: (0, 0))] * 3,
        out_shape=[jax.ShapeDtypeStruct((1, 1), jnp.float32)] * 3,
    )(*cnts)

    return (*zqs, *qs, *ds, *(p.reshape(()) for p in perps))
